# Initial kernel scaffold; baseline (speedup 1.0000x reference)
import jax
import jax.numpy as jnp
from jax.experimental import pallas as pl


def kernel(src, weight):
    raise NotImplementedError("write your pallas kernel here")


# speedup vs baseline: 1.0619x; 1.0619x over previous
import functools
import jax
import jax.numpy as jnp
from jax import lax
from jax.experimental import pallas as pl
from jax.experimental.pallas import tpu as pltpu
from jax.experimental.pallas import tpu_sc as plsc

_SEQ, _BATCH, _D = 200, 4096, 32
_TOTAL = _SEQ * _BATCH
_NC, _NS, _L = 2, 16, 16
_NW = _NC * _NS
_PER_W = _TOTAL // _NW
_CHUNK = 1024
_NCHUNK = _PER_W // _CHUNK
_SCALE = 8.0

_mesh = plsc.VectorSubcoreMesh(core_axis_name="c", subcore_axis_name="s")


@functools.partial(
    pl.kernel,
    out_type=jax.ShapeDtypeStruct((_TOTAL, _D), jnp.float32),
    mesh=_mesh,
    scratch_types=[
        pltpu.VMEM((_CHUNK,), jnp.int32),
        pltpu.VMEM((_CHUNK, _D), jnp.float32),
        pltpu.SemaphoreType.DMA,
    ],
    compiler_params=pltpu.CompilerParams(use_tc_tiling_on_sc=False),
)
def _gather_scale(src_hbm, w_hbm, out_hbm, idx_v, rows_v, sem):
    wid = lax.axis_index("s") * _NC + lax.axis_index("c")
    base = wid * _PER_W

    @pl.loop(0, _NCHUNK)
    def _chunk(g):
        off = base + g * _CHUNK
        pltpu.sync_copy(src_hbm.at[pl.ds(off, _CHUNK)], idx_v)
        pltpu.async_copy(w_hbm.at[idx_v], rows_v, sem).wait()

        @plsc.parallel_loop(0, _CHUNK, unroll=8)
        def _scale(r):
            rows_v[r, pl.ds(0, _L)] = rows_v[r, pl.ds(0, _L)] * _SCALE
            rows_v[r, pl.ds(_L, _L)] = rows_v[r, pl.ds(_L, _L)] * _SCALE

        pltpu.sync_copy(rows_v, out_hbm.at[pl.ds(off, _CHUNK)])


def kernel(src, weight):
    flat = src.reshape(_TOTAL)
    out = _gather_scale(flat, weight)
    return out.reshape(_SEQ, _BATCH, _D)


# speedup vs baseline: 1.1255x; 1.0599x over previous
import functools
import jax
import jax.numpy as jnp
from jax import lax
from jax.experimental import pallas as pl
from jax.experimental.pallas import tpu as pltpu
from jax.experimental.pallas import tpu_sc as plsc

_SEQ, _BATCH, _D = 200, 4096, 32
_TOTAL = _SEQ * _BATCH
_NC, _NS, _L = 2, 16, 16
_NW = _NC * _NS
_PER_W = _TOTAL // _NW
_CHUNK = 800
_NCHUNK = _PER_W // _CHUNK
_NBUF = 2
_SCALE = 8.0

_mesh = plsc.VectorSubcoreMesh(core_axis_name="c", subcore_axis_name="s")


@functools.partial(
    pl.kernel,
    out_type=jax.ShapeDtypeStruct((_TOTAL, _D), jnp.float32),
    mesh=_mesh,
    scratch_types=[
        pltpu.VMEM((_PER_W,), jnp.int32),
        [pltpu.VMEM((_CHUNK, _D), jnp.float32) for _ in range(_NBUF)],
        [pltpu.VMEM((_CHUNK, _D), jnp.float32) for _ in range(_NBUF)],
        [pltpu.SemaphoreType.DMA for _ in range(_NBUF)],
        [pltpu.SemaphoreType.DMA for _ in range(_NBUF)],
    ],
    compiler_params=pltpu.CompilerParams(use_tc_tiling_on_sc=False),
)
def _gather_scale(src_hbm, w_hbm, out_hbm, idx_v, ins, outs, gsems, osems):
    wid = lax.axis_index("s") * _NC + lax.axis_index("c")
    base = wid * _PER_W

    pltpu.sync_copy(src_hbm.at[pl.ds(base, _PER_W)], idx_v)

    def gather(g, b):
        src = w_hbm.at[idx_v.at[pl.ds(g * _CHUNK, _CHUNK)]]
        return pltpu.make_async_copy(src, ins[b], gsems[b])

    def writeback(g, b):
        dst = out_hbm.at[pl.ds(base + g * _CHUNK, _CHUNK)]
        return pltpu.make_async_copy(outs[b], dst, osems[b])

    for b in range(_NBUF):
        gather(b, b).start()

    @pl.loop(0, _NCHUNK, step=_NBUF)
    def _outer(g0):
        for b in range(_NBUF):
            g = g0 + b
            gather(g, b).wait()

            @pl.when(g0 >= _NBUF)
            def _():
                writeback(g, b).wait()

            @plsc.parallel_loop(0, _CHUNK, unroll=8)
            def _scale(r):
                outs[b][r, pl.ds(0, _L)] = ins[b][r, pl.ds(0, _L)] * _SCALE
                outs[b][r, pl.ds(_L, _L)] = ins[b][r, pl.ds(_L, _L)] * _SCALE

            @pl.when(g0 < _NCHUNK - _NBUF)
            def _():
                gather(g + _NBUF, b).start()

            writeback(g, b).start()

    for b in range(_NBUF):
        writeback(_NCHUNK - _NBUF + b, b).wait()


def kernel(src, weight):
    flat = src.reshape(_TOTAL)
    out = _gather_scale(flat, weight)
    return out.reshape(_SEQ, _BATCH, _D)


# speedup vs baseline: 1.4044x; 1.2478x over previous
import functools
import jax
import jax.numpy as jnp
from jax import lax
from jax.experimental import pallas as pl
from jax.experimental.pallas import tpu as pltpu
from jax.experimental.pallas import tpu_sc as plsc

_SEQ, _BATCH, _D = 200, 4096, 32
_TOTAL = _SEQ * _BATCH
_V = 1_000_000
_NC, _NS, _L = 2, 16, 16
_NW = _NC * _NS
_SCALE = 8.0

_mesh = plsc.VectorSubcoreMesh(core_axis_name="c", subcore_axis_name="s")

_NWIN = 7813
_VPAD = _NWIN * 128
_W_BASE = _NWIN // _NW
_W_EXTRA = _NWIN % _NW


@functools.partial(
    pl.kernel,
    out_type=jax.ShapeDtypeStruct((_VPAD * _D,), jnp.float32),
    mesh=_mesh,
    scratch_types=[
        [pltpu.VMEM((_D, 128), jnp.float32) for _ in range(2)],
        [pltpu.VMEM((128 * _D,), jnp.float32) for _ in range(2)],
        [pltpu.SemaphoreType.DMA for _ in range(2)],
        [pltpu.SemaphoreType.DMA for _ in range(2)],
    ],
    compiler_params=pltpu.CompilerParams(
        use_tc_tiling_on_sc=True, needs_layout_passes=False
    ),
)
def _format_table(wt_hbm, tab_hbm, tbufs, stages, isems, osems):
    wid = lax.axis_index("s") * _NC + lax.axis_index("c")
    nwin = _W_BASE + jnp.where(wid < _W_EXTRA, 1, 0)
    win0 = wid * _W_BASE + jnp.minimum(wid, _W_EXTRA)

    def col0(i):
        return pl.multiple_of((win0 + i) * 128, 128)

    def tile_in(i, b):
        return pltpu.make_async_copy(
            wt_hbm.at[:, pl.ds(col0(i), 128)], tbufs[b], isems[b]
        )

    def row_out(i, b):
        return pltpu.make_async_copy(
            stages[b], tab_hbm.at[pl.ds(col0(i) * _D, 128 * _D)], osems[b]
        )

    iota = lax.iota(jnp.int32, _L)

    def transpose_scale(b):
        @plsc.parallel_loop(0, 128, unroll=4)
        def _tr(c):
            csplat = jnp.broadcast_to(c, (_L,)).astype(jnp.int32)
            for h in range(2):
                v = plsc.load_gather(tbufs[b], [iota + h * _L, csplat])
                stages[b][pl.ds(c * _D + h * _L, _L)] = v * _SCALE

    for b in range(2):
        tile_in(b, b).start()

    @pl.loop(0, _W_BASE, step=2)
    def _win(g0):
        for b in range(2):
            g = g0 + b
            tile_in(g, b).wait()

            @pl.when(g0 >= 2)
            def _():
                row_out(g - 2, b).wait()

            transpose_scale(b)

            @pl.when(g + 2 < nwin)
            def _():
                tile_in(g + 2, b).start()

            row_out(g, b).start()

    for b in range(2):
        row_out(_W_BASE - 2 + b, b).wait()

    @pl.when(nwin > _W_BASE)
    def _tail():
        tile_in(_W_BASE, 0).wait()
        transpose_scale(0)
        row_out(_W_BASE, 0).start()
        row_out(_W_BASE, 0).wait()


_GPW = _TOTAL // 128 // _NW
_PER_W = _TOTAL // _NW


@functools.partial(
    pl.kernel,
    out_type=jax.ShapeDtypeStruct((_SEQ, 4, _BATCH // 128, 8, 128), jnp.float32),
    mesh=_mesh,
    scratch_types=[
        pltpu.VMEM((_PER_W,), jnp.int32),
        [pltpu.VMEM((128, _D), jnp.float32) for _ in range(2)],
        [pltpu.VMEM((4, 8, 128), jnp.float32) for _ in range(2)],
        [pltpu.SemaphoreType.DMA for _ in range(2)],
        [pltpu.SemaphoreType.DMA for _ in range(2)],
    ],
    compiler_params=pltpu.CompilerParams(
        use_tc_tiling_on_sc=False, needs_layout_passes=False
    ),
)
def _lookup(idx_hbm, tab_hbm, out_hbm, idx_v, rows, obufs, gsems, osems):
    wid = lax.axis_index("s") * _NC + lax.axis_index("c")
    base = wid * _PER_W
    g0 = wid * _GPW

    pltpu.sync_copy(idx_hbm.at[pl.ds(base, _PER_W)], idx_v)

    def gather(g, b):
        src = tab_hbm.at[idx_v.at[pl.ds(g * 128, 128)]]
        return pltpu.make_async_copy(src, rows[b], gsems[b])

    def writeback(g, b):
        gg = g0 + g
        dst = out_hbm.at[gg // (_BATCH // 128), :, gg % (_BATCH // 128)]
        return pltpu.make_async_copy(obufs[b], dst, osems[b])

    iota = lax.iota(jnp.int32, _L)

    for b in range(2):
        gather(b, b).start()

    @pl.loop(0, _GPW, step=2)
    def _grp(gg0):
        for b in range(2):
            g = gg0 + b
            gather(g, b).wait()

            @pl.when(gg0 >= 2)
            def _():
                writeback(g - 2, b).wait()

            @plsc.parallel_loop(0, _D, unroll=4)
            def _tr(d):
                dsplat = jnp.broadcast_to(d, (_L,)).astype(jnp.int32)
                for q in range(8):
                    v = plsc.load_gather(rows[b], [iota + q * _L, dsplat])
                    obufs[b][d // 8, d % 8, pl.ds(q * _L, _L)] = v

            @pl.when(gg0 < _GPW - 2)
            def _():
                gather(g + 2, b).start()

            writeback(g, b).start()

    for b in range(2):
        writeback(_GPW - 2 + b, b).wait()


def kernel(src, weight):
    tab = _format_table(weight.T)
    flat = src.reshape(_TOTAL)
    out5 = _lookup(flat, tab.reshape(_VPAD, _D))
    return out5.transpose(0, 2, 4, 1, 3).reshape(_SEQ, _BATCH, _D)
